# baseline (speedup 1.0000x reference)
import functools

import jax
import jax.numpy as jnp
from jax import lax
from jax.experimental import pallas as pl
from jax.experimental.pallas import tpu as pltpu
from jax.experimental.pallas import tpu_sc as plsc

N = 10000
E = 320000
F_IN = 128
H = 7
C = 10
G = 64

NC = 2
NS = 16
L = 16
HP = 8
HH = HP // NC
NP = 10240
NSL = NP // NS
NR = NP // 128

CHE = 2560
NCHT = E // CHE
FULL = NCHT // NS
REM = NCHT - FULL * NS
U = 5


def _sc_mesh():
    return plsc.VectorSubcoreMesh(core_axis_name="c", subcore_axis_name="s",
                                  num_cores=NC, num_subcores=NS)


_SC_PARAMS = pltpu.CompilerParams(needs_layout_passes=False)


PB = NP // 8


def _proj_body(x_ref, wT_ref, hT_ref):
    hT_ref[...] = lax.dot_general(
        wT_ref[...], x_ref[...], (((1,), (1,)), ((), ())),
        preferred_element_type=jnp.float32)


def _proj(x, wT):
    return pl.pallas_call(
        _proj_body,
        grid=(NP // PB,),
        in_specs=[
            pl.BlockSpec((PB, F_IN), lambda i: (i, 0)),
            pl.BlockSpec((HP, F_IN), lambda i: (0, 0)),
        ],
        out_specs=pl.BlockSpec((HP, PB), lambda i: (0, i)),
        out_shape=jax.ShapeDtypeStruct((HP, NP), jnp.float32),
    )(x, wT)


def _fused_body(hT_hbm, ei_hbm, zeros_hbm,
                i0_hbm, i1_hbm, i2_hbm, i3_hbm,
                st_hbm, dinv_hbm,
                ebuf0_v, ebuf1_v, ebuf2_v, ebuf3_v,
                acc0_v, acc1_v, acc2_v, acc3_v, tab_v, dinv_v,
                ntmp_v, ntmp2_v,
                i0_v, i1_v, i2_v, i3_v, sdeg_s, sdinv_s, sacc_s,
                sem_s, sem_d):
    c = lax.axis_index("c")
    s = lax.axis_index("s")
    chh = pl.multiple_of(c * HH, HH)
    accs = [acc0_v, acc1_v, acc2_v, acc3_v]
    idxs = [i0_v, i1_v, i2_v, i3_v]
    ebufs = [ebuf0_v, ebuf1_v, ebuf2_v, ebuf3_v]
    NB = len(ebufs)
    has_extra = s < REM

    def start_e(i, buf):
        cb = pl.multiple_of((s + i * NS) * CHE, CHE)
        return pltpu.async_copy(
            ei_hbm.at[pl.ds(0, 2), pl.ds(cb, CHE)], ebufs[buf], sem_d)

    with jax.named_scope("ph_stage"):
        early = [
            pltpu.async_copy(i0_hbm, i0_v, sem_s),
            pltpu.async_copy(i1_hbm, i1_v, sem_s),
            pltpu.async_copy(i2_hbm, i2_v, sem_s),
            pltpu.async_copy(i3_hbm, i3_v, sem_s),
            pltpu.async_copy(zeros_hbm.at[pl.ds(0, NR)], acc0_v, sem_s),
        ]
        late = [
            pltpu.async_copy(hT_hbm.at[pl.ds(chh, HH)], tab_v, sem_s),
        ] + [
            pltpu.async_copy(zeros_hbm.at[pl.ds(0, NR)], a, sem_s)
            for a in accs[1:]
        ]
        zdeg = pltpu.make_async_copy(zeros_hbm.at[pl.ds(0, NR)], sdeg_s,
                                     sem_s)
        zsacc = pltpu.make_async_copy(zeros_hbm, sacc_s, sem_s)

        @pl.when(s == 0)
        def _():
            zdeg.start()
            zsacc.start()

        for cp in early:
            cp.wait()

    ones = jnp.full((L,), 1.0, jnp.float32)

    def hbody(buf):
        def body(i, carry):
            dvs = [ebufs[buf][1, pl.ds((i * U + u) * L, L)]
                   for u in range(U)]
            for dv in dvs:
                rv = lax.shift_right_logical(dv, 7)
                cv = lax.bitwise_and(dv, 127)
                plsc.addupdate_scatter(acc0_v, [rv, cv], ones)
            return carry
        lax.fori_loop(0, CHE // (L * U), body, 0)

    def run_chunks(process):
        depth = NB - 1
        xb = FULL % NB
        xcp = pltpu.make_async_copy(
            ei_hbm.at[pl.ds(0, 2),
                      pl.ds(pl.multiple_of((s + FULL * NS) * CHE, CHE),
                            CHE)],
            ebufs[xb], sem_d)
        cps = {i: start_e(i, i % NB) for i in range(min(depth, FULL))}
        for k in range(FULL):
            cps[k].wait()
            nxt = k + depth
            if nxt < FULL:
                cps[nxt] = start_e(nxt, nxt % NB)
            elif nxt == FULL:
                @pl.when(has_extra)
                def _():
                    xcp.start()
            process(k % NB)

        @pl.when(has_extra)
        def _():
            xcp.wait()

        @pl.when(has_extra)
        def _():
            process(xb)

    with jax.named_scope("ph_hist"):
        run_chunks(hbody)

    with jax.named_scope("ph_degmerge"):
        for cp in late:
            cp.wait()

        @pl.when(s == 0)
        def _():
            zdeg.wait()
            zsacc.wait()

        plsc.subcore_barrier()
        pltpu.sync_copy(acc0_v, sdeg_s.at[i0_v], add=True)
        plsc.subcore_barrier()

    nsl = pl.multiple_of(s * NSL, NSL)
    nrs = pl.multiple_of(s * (NR // NS), NR // NS)
    pltpu.sync_copy(sdeg_s.at[pl.ds(nrs, NR // NS)], ntmp_v)

    for r in range(NR // NS):
        for kk in range(128 // L):
            d = ntmp_v[r, pl.ds(kk * L, L)] + 1.0
            ii = plsc.bitcast(d, jnp.int32)
            ii = 0x5F3759DF - lax.shift_right_logical(ii, 1)
            y = plsc.bitcast(ii, jnp.float32)
            hd = 0.5 * d
            for _ in range(3):
                y = y * (1.5 - hd * y * y)
            ntmp2_v[0, pl.ds(r * 128 + kk * L, L)] = y

    with jax.named_scope("ph_newton"):
        pltpu.sync_copy(ntmp2_v, sdinv_s.at[pl.ds(0, 1), pl.ds(nsl, NSL)])

        @pl.when(c == 0)
        def _():
            pltpu.sync_copy(ntmp2_v,
                            dinv_hbm.at[pl.ds(0, 1), pl.ds(nsl, NSL)])

        plsc.subcore_barrier()
        pltpu.sync_copy(sdinv_s, dinv_v)

    def tsbody(i, carry):
        sls = [pl.ds((i * U + u) * L, L) for u in range(U)]
        dvs = [dinv_v[0, sl] for sl in sls]
        ts = [[tab_v[j, sl] for j in range(HH)] for sl in sls]
        ps = [[ts[u][j] * dvs[u] for j in range(HH)] for u in range(U)]
        for u in range(U):
            for j in range(HH):
                tab_v[j, sls[u]] = ps[u][j]
        return carry

    with jax.named_scope("ph_tabscale"):
        rz = pltpu.async_copy(zeros_hbm.at[pl.ds(0, NR)], acc0_v, sem_s)
        lax.fori_loop(0, NP // (L * U), tsbody, 0)
        rz.wait()
    jvs = [jnp.full((L,), j, jnp.int32) for j in range(HH)]

    def mbody(buf):
        def body(i, carry):
            svs = [ebufs[buf][0, pl.ds((i * U + u) * L, L)]
                   for u in range(U)]
            dvs = [ebufs[buf][1, pl.ds((i * U + u) * L, L)]
                   for u in range(U)]
            vals = [[plsc.load_gather(tab_v, [jvs[j], svs[u]])
                     for j in range(HH)] for u in range(U)]
            rvs = [lax.shift_right_logical(dv, 7) for dv in dvs]
            cvs = [lax.bitwise_and(dv, 127) for dv in dvs]
            for u in range(U):
                for j in range(HH):
                    plsc.addupdate_scatter(accs[j], [rvs[u], cvs[u]],
                                           vals[u][j])
            return carry
        lax.fori_loop(0, CHE // (L * U), body, 0)

    with jax.named_scope("ph_msg"):
        run_chunks(mbody)

    with jax.named_scope("ph_accmerge"):
        for j in range(HH):
            pltpu.sync_copy(accs[j], sacc_s.at[idxs[j]], add=True)
        plsc.subcore_barrier()

    with jax.named_scope("ph_out"):
        @pl.when(s == 0)
        def _():
            pltpu.sync_copy(sacc_s, st_hbm.at[c])


def _fused(hT, ei, zeros, i0, i1, i2, i3):
    return pl.kernel(
        _fused_body,
        out_type=[
            jax.ShapeDtypeStruct((NC, HH * NR, 128), jnp.float32),
            jax.ShapeDtypeStruct((1, NP), jnp.float32),
        ],
        mesh=_sc_mesh(),
        compiler_params=_SC_PARAMS,
        scratch_types=[
            pltpu.VMEM((2, CHE), jnp.int32),
            pltpu.VMEM((2, CHE), jnp.int32),
            pltpu.VMEM((2, CHE), jnp.int32),
            pltpu.VMEM((2, CHE), jnp.int32),
            pltpu.VMEM((NR, 128), jnp.float32),
            pltpu.VMEM((NR, 128), jnp.float32),
            pltpu.VMEM((NR, 128), jnp.float32),
            pltpu.VMEM((NR, 128), jnp.float32),
            pltpu.VMEM((HH, NP), jnp.float32),
            pltpu.VMEM((1, NP), jnp.float32),
            pltpu.VMEM((NR // NS, 128), jnp.float32),
            pltpu.VMEM((1, NSL), jnp.float32),
            pltpu.VMEM((NR,), jnp.int32),
            pltpu.VMEM((NR,), jnp.int32),
            pltpu.VMEM((NR,), jnp.int32),
            pltpu.VMEM((NR,), jnp.int32),
            pltpu.VMEM_SHARED((NR, 128), jnp.float32),
            pltpu.VMEM_SHARED((1, NP), jnp.float32),
            pltpu.VMEM_SHARED((HH * NR, 128), jnp.float32),
            pltpu.SemaphoreType.DMA,
            pltpu.SemaphoreType.DMA,
        ],
    )(hT, ei, zeros, i0, i1, i2, i3)


def _final_body(st_ref, hT_ref, dinv_ref, bin_ref, bi_ref,
                woutT_ref, bout_ref, out_ref):
    dinvT = dinv_ref[...][:, :N]
    st = st_ref[...][:, :N]
    hT = hT_ref[...][:, :N]
    outT = jnp.maximum(dinvT * (st + hT * dinvT) + bin_ref[...], 0.0)
    gids = lax.broadcasted_iota(jnp.int32, (G, N), 0)
    onehot = jnp.where(gids == bi_ref[...], 1.0, 0.0)
    pooledT = lax.dot_general(
        outT, onehot, (((1,), (1,)), ((), ())),
        preferred_element_type=jnp.float32)
    ones_row = jnp.ones((1, N), jnp.float32)
    counts = lax.dot_general(
        ones_row, onehot, (((1,), (1,)), ((), ())),
        preferred_element_type=jnp.float32)
    pooledT = pooledT / jnp.maximum(counts, 1.0)
    logitsT = jnp.dot(woutT_ref[...], pooledT,
                      preferred_element_type=jnp.float32) + bout_ref[...]
    m = jnp.max(logitsT, axis=0, keepdims=True)
    e = jnp.exp(logitsT - m)
    out_ref[...] = e / jnp.sum(e, axis=0, keepdims=True)


def _final(st, hT, dinvT, bin_col, bi_row, woutT, bout_col):
    return pl.pallas_call(
        _final_body,
        out_shape=jax.ShapeDtypeStruct((C, G), jnp.float32),
    )(st, hT, dinvT, bin_col, bi_row, woutT, bout_col)


def kernel(x, edge_index, batch_index, W_in, b_in, W1, b1, W_out, b_out):
    zeros = jnp.zeros((HH * NR, 128), jnp.float32)
    idxs = [jnp.arange(NR, dtype=jnp.int32) + NR * j for j in range(HH)]

    wT = jnp.zeros((HP, F_IN), jnp.float32).at[:H].set(W_in.T)
    woutT = jnp.zeros((C, HP), jnp.float32).at[:, :H].set(W_out.T)
    bin_col = jnp.zeros((HP, 1), jnp.float32).at[:H, 0].set(b_in)
    bout_col = b_out.reshape(C, 1)
    bi_row = batch_index.reshape(1, N)

    hT = _proj(x, wT)
    st, dinvT = _fused(hT, edge_index, zeros, *idxs)
    st8 = st.reshape(NC, HH, NP).reshape(NC * HH, NP)
    probsT = _final(st8, hT, dinvT, bin_col, bi_row, woutT, bout_col)
    return probsT.T


# speedup vs baseline: 1.0401x; 1.0401x over previous
import functools

import jax
import jax.numpy as jnp
from jax import lax
from jax.experimental import pallas as pl
from jax.experimental.pallas import tpu as pltpu
from jax.experimental.pallas import tpu_sc as plsc

N = 10000
E = 320000
F_IN = 128
H = 7
C = 10
G = 64

NC = 2
NS = 16
L = 16
HP = 8
HH = HP // NC
NP = 10240
NSL = NP // NS
NR = NP // 128

CHE = 2560
NCHT = E // CHE
FULL = NCHT // NS
REM = NCHT - FULL * NS
U = 5


def _sc_mesh():
    return plsc.VectorSubcoreMesh(core_axis_name="c", subcore_axis_name="s",
                                  num_cores=NC, num_subcores=NS)


_SC_PARAMS = pltpu.CompilerParams(needs_layout_passes=False)


def _proj_body(x_ref, wT_ref, hT_ref):
    hT_ref[:, :N] = lax.dot_general(
        wT_ref[...], x_ref[...], (((1,), (1,)), ((), ())),
        preferred_element_type=jnp.float32)


def _proj(x, wT):
    return pl.pallas_call(
        _proj_body,
        out_shape=jax.ShapeDtypeStruct((HP, NP), jnp.float32),
    )(x, wT)


def _fused_body(hT_hbm, ei_hbm, zeros_hbm,
                i0_hbm, i1_hbm, i2_hbm, i3_hbm,
                st_hbm, dinv_hbm,
                ebuf0_v, ebuf1_v, ebuf2_v, ebuf3_v,
                acc0_v, acc1_v, acc2_v, acc3_v, tab_v, dinv_v,
                ntmp_v, ntmp2_v,
                i0_v, i1_v, i2_v, i3_v, sdeg_s, sdinv_s, sacc_s,
                sem_s, sem_d):
    c = lax.axis_index("c")
    s = lax.axis_index("s")
    chh = pl.multiple_of(c * HH, HH)
    accs = [acc0_v, acc1_v, acc2_v, acc3_v]
    idxs = [i0_v, i1_v, i2_v, i3_v]
    ebufs = [ebuf0_v, ebuf1_v, ebuf2_v, ebuf3_v]
    NB = len(ebufs)
    has_extra = s < REM

    def start_e(i, buf):
        cb = pl.multiple_of((s + i * NS) * CHE, CHE)
        return pltpu.async_copy(
            ei_hbm.at[pl.ds(0, 2), pl.ds(cb, CHE)], ebufs[buf], sem_d)

    with jax.named_scope("ph_stage"):
        early = [
            pltpu.async_copy(i0_hbm, i0_v, sem_s),
            pltpu.async_copy(i1_hbm, i1_v, sem_s),
            pltpu.async_copy(i2_hbm, i2_v, sem_s),
            pltpu.async_copy(i3_hbm, i3_v, sem_s),
            pltpu.async_copy(zeros_hbm.at[pl.ds(0, NR)], acc0_v, sem_s),
        ]
        late = [
            pltpu.async_copy(hT_hbm.at[pl.ds(chh, HH)], tab_v, sem_s),
        ] + [
            pltpu.async_copy(zeros_hbm.at[pl.ds(0, NR)], a, sem_s)
            for a in accs[1:]
        ]
        zdeg = pltpu.make_async_copy(zeros_hbm.at[pl.ds(0, NR)], sdeg_s,
                                     sem_s)
        zsacc = pltpu.make_async_copy(zeros_hbm, sacc_s, sem_s)

        @pl.when(s == 0)
        def _():
            zdeg.start()
            zsacc.start()

        for cp in early:
            cp.wait()

    ones = jnp.full((L,), 1.0, jnp.float32)

    def hbody(buf):
        def body(i, carry):
            dvs = [ebufs[buf][1, pl.ds((i * U + u) * L, L)]
                   for u in range(U)]
            for dv in dvs:
                rv = lax.shift_right_logical(dv, 7)
                cv = lax.bitwise_and(dv, 127)
                plsc.addupdate_scatter(acc0_v, [rv, cv], ones)
            return carry
        lax.fori_loop(0, CHE // (L * U), body, 0)

    def run_chunks(process):
        depth = NB - 1
        xb = FULL % NB
        xcp = pltpu.make_async_copy(
            ei_hbm.at[pl.ds(0, 2),
                      pl.ds(pl.multiple_of((s + FULL * NS) * CHE, CHE),
                            CHE)],
            ebufs[xb], sem_d)
        cps = {i: start_e(i, i % NB) for i in range(min(depth, FULL))}
        for k in range(FULL):
            cps[k].wait()
            nxt = k + depth
            if nxt < FULL:
                cps[nxt] = start_e(nxt, nxt % NB)
            elif nxt == FULL:
                @pl.when(has_extra)
                def _():
                    xcp.start()
            process(k % NB)

        @pl.when(has_extra)
        def _():
            xcp.wait()

        @pl.when(has_extra)
        def _():
            process(xb)

    with jax.named_scope("ph_hist"):
        run_chunks(hbody)

    with jax.named_scope("ph_degmerge"):
        for cp in late:
            cp.wait()

        @pl.when(s == 0)
        def _():
            zdeg.wait()
            zsacc.wait()

        plsc.subcore_barrier()
        pltpu.sync_copy(acc0_v, sdeg_s.at[i0_v], add=True)
        plsc.subcore_barrier()

    nsl = pl.multiple_of(s * NSL, NSL)
    nrs = pl.multiple_of(s * (NR // NS), NR // NS)
    pltpu.sync_copy(sdeg_s.at[pl.ds(nrs, NR // NS)], ntmp_v)

    for r in range(NR // NS):
        for kk in range(128 // L):
            d = ntmp_v[r, pl.ds(kk * L, L)] + 1.0
            ii = plsc.bitcast(d, jnp.int32)
            ii = 0x5F3759DF - lax.shift_right_logical(ii, 1)
            y = plsc.bitcast(ii, jnp.float32)
            hd = 0.5 * d
            for _ in range(3):
                y = y * (1.5 - hd * y * y)
            ntmp2_v[0, pl.ds(r * 128 + kk * L, L)] = y

    with jax.named_scope("ph_newton"):
        pltpu.sync_copy(ntmp2_v, sdinv_s.at[pl.ds(0, 1), pl.ds(nsl, NSL)])

        @pl.when(c == 0)
        def _():
            pltpu.sync_copy(ntmp2_v,
                            dinv_hbm.at[pl.ds(0, 1), pl.ds(nsl, NSL)])

        plsc.subcore_barrier()
        pltpu.sync_copy(sdinv_s, dinv_v)

    def tsbody(i, carry):
        sls = [pl.ds((i * U + u) * L, L) for u in range(U)]
        dvs = [dinv_v[0, sl] for sl in sls]
        ts = [[tab_v[j, sl] for j in range(HH)] for sl in sls]
        ps = [[ts[u][j] * dvs[u] for j in range(HH)] for u in range(U)]
        for u in range(U):
            for j in range(HH):
                tab_v[j, sls[u]] = ps[u][j]
        return carry

    with jax.named_scope("ph_tabscale"):
        rz = pltpu.async_copy(zeros_hbm.at[pl.ds(0, NR)], acc0_v, sem_s)
        lax.fori_loop(0, NP // (L * U), tsbody, 0)
        rz.wait()
    jvs = [jnp.full((L,), j, jnp.int32) for j in range(HH)]

    def mbody(buf):
        def body(i, carry):
            svs = [ebufs[buf][0, pl.ds((i * U + u) * L, L)]
                   for u in range(U)]
            dvs = [ebufs[buf][1, pl.ds((i * U + u) * L, L)]
                   for u in range(U)]
            vals = [[plsc.load_gather(tab_v, [jvs[j], svs[u]])
                     for j in range(HH)] for u in range(U)]
            rvs = [lax.shift_right_logical(dv, 7) for dv in dvs]
            cvs = [lax.bitwise_and(dv, 127) for dv in dvs]
            for u in range(U):
                for j in range(HH):
                    plsc.addupdate_scatter(accs[j], [rvs[u], cvs[u]],
                                           vals[u][j])
            return carry
        lax.fori_loop(0, CHE // (L * U), body, 0)

    with jax.named_scope("ph_msg"):
        run_chunks(mbody)

    with jax.named_scope("ph_accmerge"):
        for j in range(HH):
            pltpu.sync_copy(accs[j], sacc_s.at[idxs[j]], add=True)
        plsc.subcore_barrier()

    with jax.named_scope("ph_out"):
        @pl.when(s == 0)
        def _():
            pltpu.sync_copy(sacc_s, st_hbm.at[c])


def _fused(hT, ei, zeros, i0, i1, i2, i3):
    return pl.kernel(
        _fused_body,
        out_type=[
            jax.ShapeDtypeStruct((NC, HH * NR, 128), jnp.float32),
            jax.ShapeDtypeStruct((1, NP), jnp.float32),
        ],
        mesh=_sc_mesh(),
        compiler_params=_SC_PARAMS,
        scratch_types=[
            pltpu.VMEM((2, CHE), jnp.int32),
            pltpu.VMEM((2, CHE), jnp.int32),
            pltpu.VMEM((2, CHE), jnp.int32),
            pltpu.VMEM((2, CHE), jnp.int32),
            pltpu.VMEM((NR, 128), jnp.float32),
            pltpu.VMEM((NR, 128), jnp.float32),
            pltpu.VMEM((NR, 128), jnp.float32),
            pltpu.VMEM((NR, 128), jnp.float32),
            pltpu.VMEM((HH, NP), jnp.float32),
            pltpu.VMEM((1, NP), jnp.float32),
            pltpu.VMEM((NR // NS, 128), jnp.float32),
            pltpu.VMEM((1, NSL), jnp.float32),
            pltpu.VMEM((NR,), jnp.int32),
            pltpu.VMEM((NR,), jnp.int32),
            pltpu.VMEM((NR,), jnp.int32),
            pltpu.VMEM((NR,), jnp.int32),
            pltpu.VMEM_SHARED((NR, 128), jnp.float32),
            pltpu.VMEM_SHARED((1, NP), jnp.float32),
            pltpu.VMEM_SHARED((HH * NR, 128), jnp.float32),
            pltpu.SemaphoreType.DMA,
            pltpu.SemaphoreType.DMA,
        ],
    )(hT, ei, zeros, i0, i1, i2, i3)


def _final_body(st_ref, hT_ref, dinv_ref, bin_ref, bi_ref,
                woutT_ref, bout_ref, out_ref):
    dinvT = dinv_ref[...][:, :N]
    st = st_ref[...][:, :N]
    hT = hT_ref[...][:, :N]
    outT = jnp.maximum(dinvT * (st + hT * dinvT) + bin_ref[...], 0.0)
    gids = lax.broadcasted_iota(jnp.int32, (G, N), 0)
    onehot = jnp.where(gids == bi_ref[...], 1.0, 0.0)
    pooledT = lax.dot_general(
        outT, onehot, (((1,), (1,)), ((), ())),
        preferred_element_type=jnp.float32)
    ones_row = jnp.ones((1, N), jnp.float32)
    counts = lax.dot_general(
        ones_row, onehot, (((1,), (1,)), ((), ())),
        preferred_element_type=jnp.float32)
    pooledT = pooledT / jnp.maximum(counts, 1.0)
    logitsT = jnp.dot(woutT_ref[...], pooledT,
                      preferred_element_type=jnp.float32) + bout_ref[...]
    m = jnp.max(logitsT, axis=0, keepdims=True)
    e = jnp.exp(logitsT - m)
    out_ref[...] = e / jnp.sum(e, axis=0, keepdims=True)


def _final(st, hT, dinvT, bin_col, bi_row, woutT, bout_col):
    return pl.pallas_call(
        _final_body,
        out_shape=jax.ShapeDtypeStruct((C, G), jnp.float32),
    )(st, hT, dinvT, bin_col, bi_row, woutT, bout_col)


def kernel(x, edge_index, batch_index, W_in, b_in, W1, b1, W_out, b_out):
    zeros = jnp.zeros((HH * NR, 128), jnp.float32)
    idxs = [jnp.arange(NR, dtype=jnp.int32) + NR * j for j in range(HH)]

    wT = jnp.zeros((HP, F_IN), jnp.float32).at[:H].set(W_in.T)
    woutT = jnp.zeros((C, HP), jnp.float32).at[:, :H].set(W_out.T)
    bin_col = jnp.zeros((HP, 1), jnp.float32).at[:H, 0].set(b_in)
    bout_col = b_out.reshape(C, 1)
    bi_row = batch_index.reshape(1, N)

    hT = _proj(x, wT)
    st, dinvT = _fused(hT, edge_index, zeros, *idxs)
    st8 = st.reshape(NC, HH, NP).reshape(NC * HH, NP)
    probsT = _final(st8, hT, dinvT, bin_col, bi_row, woutT, bout_col)
    return probsT.T


# speedup vs baseline: 1.0883x; 1.0463x over previous
import functools

import jax
import jax.numpy as jnp
from jax import lax
from jax.experimental import pallas as pl
from jax.experimental.pallas import tpu as pltpu
from jax.experimental.pallas import tpu_sc as plsc

N = 10000
E = 320000
F_IN = 128
H = 7
C = 10
G = 64

NC = 2
NS = 16
L = 16
HP = 8
HH = HP // NC
NP = 10240
NSL = NP // NS
NR = NP // 128

CHE = 2560
NCHT = E // CHE
FULL = NCHT // NS
REM = NCHT - FULL * NS
U = 5


def _sc_mesh():
    return plsc.VectorSubcoreMesh(core_axis_name="c", subcore_axis_name="s",
                                  num_cores=NC, num_subcores=NS)


_SC_PARAMS = pltpu.CompilerParams(needs_layout_passes=False)


def _proj_body(x_ref, wT_ref, hT_ref):
    hT_ref[:, :N] = lax.dot_general(
        wT_ref[...], x_ref[...], (((1,), (1,)), ((), ())),
        preferred_element_type=jnp.float32)


def _proj(x, wT):
    return pl.pallas_call(
        _proj_body,
        out_shape=jax.ShapeDtypeStruct((HP, NP), jnp.float32),
    )(x, wT)


def _fused_body(hT_hbm, ei_hbm, zeros_hbm,
                i0_hbm, i1_hbm, i2_hbm, i3_hbm,
                st_hbm, dinv_hbm,
                ebuf0_v, ebuf1_v, ebuf2_v, ebuf3_v,
                acc0_v, acc1_v, acc2_v, acc3_v, tab_v, tsl_v,
                ntmp_v, ntmp2_v,
                i0_v, i1_v, i2_v, i3_v, sdeg_s, stab_s, sacc_s,
                sem_s, sem_d):
    c = lax.axis_index("c")
    s = lax.axis_index("s")
    chh = pl.multiple_of(c * HH, HH)
    accs = [acc0_v, acc1_v, acc2_v, acc3_v]
    idxs = [i0_v, i1_v, i2_v, i3_v]
    ebufs = [ebuf0_v, ebuf1_v, ebuf2_v, ebuf3_v]
    NB = len(ebufs)
    has_extra = s < REM

    def start_e(i, buf):
        cb = pl.multiple_of((s + i * NS) * CHE, CHE)
        return pltpu.async_copy(
            ei_hbm.at[pl.ds(0, 2), pl.ds(cb, CHE)], ebufs[buf], sem_d)

    with jax.named_scope("ph_stage"):
        early = [
            pltpu.async_copy(i0_hbm, i0_v, sem_s),
            pltpu.async_copy(i1_hbm, i1_v, sem_s),
            pltpu.async_copy(i2_hbm, i2_v, sem_s),
            pltpu.async_copy(i3_hbm, i3_v, sem_s),
            pltpu.async_copy(zeros_hbm.at[pl.ds(0, NR)], acc0_v, sem_s),
        ]
        late = [
            pltpu.async_copy(
                hT_hbm.at[pl.ds(chh, HH), pl.ds(pl.multiple_of(s * NSL, NSL),
                                                NSL)],
                tsl_v, sem_s),
        ] + [
            pltpu.async_copy(zeros_hbm.at[pl.ds(0, NR)], a, sem_s)
            for a in accs[1:]
        ]
        zdeg = pltpu.make_async_copy(zeros_hbm.at[pl.ds(0, NR)], sdeg_s,
                                     sem_s)
        zsacc = pltpu.make_async_copy(zeros_hbm, sacc_s, sem_s)

        @pl.when(s == 0)
        def _():
            zdeg.start()
            zsacc.start()

        for cp in early:
            cp.wait()

    ones = jnp.full((L,), 1.0, jnp.float32)

    def hbody(buf):
        def body(i, carry):
            dvs = [ebufs[buf][1, pl.ds((i * U + u) * L, L)]
                   for u in range(U)]
            for dv in dvs:
                rv = lax.shift_right_logical(dv, 7)
                cv = lax.bitwise_and(dv, 127)
                plsc.addupdate_scatter(acc0_v, [rv, cv], ones)
            return carry
        lax.fori_loop(0, CHE // (L * U), body, 0)

    def run_chunks(process):
        depth = NB - 1
        xb = FULL % NB
        xcp = pltpu.make_async_copy(
            ei_hbm.at[pl.ds(0, 2),
                      pl.ds(pl.multiple_of((s + FULL * NS) * CHE, CHE),
                            CHE)],
            ebufs[xb], sem_d)
        cps = {i: start_e(i, i % NB) for i in range(min(depth, FULL))}
        for k in range(FULL):
            cps[k].wait()
            nxt = k + depth
            if nxt < FULL:
                cps[nxt] = start_e(nxt, nxt % NB)
            elif nxt == FULL:
                @pl.when(has_extra)
                def _():
                    xcp.start()
            process(k % NB)

        @pl.when(has_extra)
        def _():
            xcp.wait()

        @pl.when(has_extra)
        def _():
            process(xb)

    with jax.named_scope("ph_hist"):
        run_chunks(hbody)

    with jax.named_scope("ph_degmerge"):
        for cp in late:
            cp.wait()

        @pl.when(s == 0)
        def _():
            zdeg.wait()
            zsacc.wait()

        plsc.subcore_barrier()
        pltpu.sync_copy(acc0_v, sdeg_s.at[i0_v], add=True)
        plsc.subcore_barrier()

    nsl = pl.multiple_of(s * NSL, NSL)
    nrs = pl.multiple_of(s * (NR // NS), NR // NS)
    pltpu.sync_copy(sdeg_s.at[pl.ds(nrs, NR // NS)], ntmp_v)

    for r in range(NR // NS):
        for kk in range(128 // L):
            d = ntmp_v[r, pl.ds(kk * L, L)] + 1.0
            ii = plsc.bitcast(d, jnp.int32)
            ii = 0x5F3759DF - lax.shift_right_logical(ii, 1)
            y = plsc.bitcast(ii, jnp.float32)
            hd = 0.5 * d
            for _ in range(3):
                y = y * (1.5 - hd * y * y)
            ntmp2_v[0, pl.ds(r * 128 + kk * L, L)] = y

    with jax.named_scope("ph_newton"):
        @pl.when(c == 0)
        def _():
            pltpu.sync_copy(ntmp2_v,
                            dinv_hbm.at[pl.ds(0, 1), pl.ds(nsl, NSL)])

    def tsbody(i, carry):
        sl = pl.ds(i * L, L)
        dv = ntmp2_v[0, sl]
        ts = [tsl_v[j, sl] for j in range(HH)]
        for j in range(HH):
            tsl_v[j, sl] = ts[j] * dv
        return carry

    with jax.named_scope("ph_tabscale"):
        rz = pltpu.async_copy(zeros_hbm.at[pl.ds(0, NR)], acc0_v, sem_s)
        lax.fori_loop(0, NSL // L, tsbody, 0)
        pltpu.sync_copy(tsl_v, stab_s.at[pl.ds(0, HH), pl.ds(nsl, NSL)])
        plsc.subcore_barrier()
        pltpu.sync_copy(stab_s, tab_v)
        rz.wait()
    jvs = [jnp.full((L,), j, jnp.int32) for j in range(HH)]

    def mbody(buf):
        def body(i, carry):
            svs = [ebufs[buf][0, pl.ds((i * U + u) * L, L)]
                   for u in range(U)]
            dvs = [ebufs[buf][1, pl.ds((i * U + u) * L, L)]
                   for u in range(U)]
            vals = [[plsc.load_gather(tab_v, [jvs[j], svs[u]])
                     for j in range(HH)] for u in range(U)]
            rvs = [lax.shift_right_logical(dv, 7) for dv in dvs]
            cvs = [lax.bitwise_and(dv, 127) for dv in dvs]
            for u in range(U):
                for j in range(HH):
                    plsc.addupdate_scatter(accs[j], [rvs[u], cvs[u]],
                                           vals[u][j])
            return carry
        lax.fori_loop(0, CHE // (L * U), body, 0)

    with jax.named_scope("ph_msg"):
        run_chunks(mbody)

    with jax.named_scope("ph_accmerge"):
        for j in range(HH):
            pltpu.sync_copy(accs[j], sacc_s.at[idxs[j]], add=True)
        plsc.subcore_barrier()

    with jax.named_scope("ph_out"):
        @pl.when(s == 0)
        def _():
            pltpu.sync_copy(sacc_s, st_hbm.at[c])


def _fused(hT, ei, zeros, i0, i1, i2, i3):
    return pl.kernel(
        _fused_body,
        out_type=[
            jax.ShapeDtypeStruct((NC, HH * NR, 128), jnp.float32),
            jax.ShapeDtypeStruct((1, NP), jnp.float32),
        ],
        mesh=_sc_mesh(),
        compiler_params=_SC_PARAMS,
        scratch_types=[
            pltpu.VMEM((2, CHE), jnp.int32),
            pltpu.VMEM((2, CHE), jnp.int32),
            pltpu.VMEM((2, CHE), jnp.int32),
            pltpu.VMEM((2, CHE), jnp.int32),
            pltpu.VMEM((NR, 128), jnp.float32),
            pltpu.VMEM((NR, 128), jnp.float32),
            pltpu.VMEM((NR, 128), jnp.float32),
            pltpu.VMEM((NR, 128), jnp.float32),
            pltpu.VMEM((HH, NP), jnp.float32),
            pltpu.VMEM((HH, NSL), jnp.float32),
            pltpu.VMEM((NR // NS, 128), jnp.float32),
            pltpu.VMEM((1, NSL), jnp.float32),
            pltpu.VMEM((NR,), jnp.int32),
            pltpu.VMEM((NR,), jnp.int32),
            pltpu.VMEM((NR,), jnp.int32),
            pltpu.VMEM((NR,), jnp.int32),
            pltpu.VMEM_SHARED((NR, 128), jnp.float32),
            pltpu.VMEM_SHARED((HH, NP), jnp.float32),
            pltpu.VMEM_SHARED((HH * NR, 128), jnp.float32),
            pltpu.SemaphoreType.DMA,
            pltpu.SemaphoreType.DMA,
        ],
    )(hT, ei, zeros, i0, i1, i2, i3)


def _final_body(st_ref, hT_ref, dinv_ref, bin_ref, bi_ref,
                woutT_ref, bout_ref, out_ref):
    dinvT = dinv_ref[...][:, :N]
    st = st_ref[...][:, :N]
    hT = hT_ref[...][:, :N]
    outT = jnp.maximum(dinvT * (st + hT * dinvT) + bin_ref[...], 0.0)
    gids = lax.broadcasted_iota(jnp.int32, (G, N), 0)
    onehot = jnp.where(gids == bi_ref[...], 1.0, 0.0)
    pooledT = lax.dot_general(
        outT, onehot, (((1,), (1,)), ((), ())),
        preferred_element_type=jnp.float32)
    ones_row = jnp.ones((1, N), jnp.float32)
    counts = lax.dot_general(
        ones_row, onehot, (((1,), (1,)), ((), ())),
        preferred_element_type=jnp.float32)
    pooledT = pooledT / jnp.maximum(counts, 1.0)
    logitsT = jnp.dot(woutT_ref[...], pooledT,
                      preferred_element_type=jnp.float32) + bout_ref[...]
    m = jnp.max(logitsT, axis=0, keepdims=True)
    e = jnp.exp(logitsT - m)
    out_ref[...] = e / jnp.sum(e, axis=0, keepdims=True)


def _final(st, hT, dinvT, bin_col, bi_row, woutT, bout_col):
    return pl.pallas_call(
        _final_body,
        out_shape=jax.ShapeDtypeStruct((C, G), jnp.float32),
    )(st, hT, dinvT, bin_col, bi_row, woutT, bout_col)


def kernel(x, edge_index, batch_index, W_in, b_in, W1, b1, W_out, b_out):
    zeros = jnp.zeros((HH * NR, 128), jnp.float32)
    idxs = [jnp.arange(NR, dtype=jnp.int32) + NR * j for j in range(HH)]

    wT = jnp.zeros((HP, F_IN), jnp.float32).at[:H].set(W_in.T)
    woutT = jnp.zeros((C, HP), jnp.float32).at[:, :H].set(W_out.T)
    bin_col = jnp.zeros((HP, 1), jnp.float32).at[:H, 0].set(b_in)
    bout_col = b_out.reshape(C, 1)
    bi_row = batch_index.reshape(1, N)

    hT = _proj(x, wT)
    st, dinvT = _fused(hT, edge_index, zeros, *idxs)
    st8 = st.reshape(NC, HH, NP).reshape(NC * HH, NP)
    probsT = _final(st8, hT, dinvT, bin_col, bi_row, woutT, bout_col)
    return probsT.T
